# baseline (speedup 1.0000x reference)
import functools

import jax
import jax.numpy as jnp
from jax import lax
from jax.experimental import pallas as pl
from jax.experimental.pallas import tpu as pltpu
from jax.experimental.pallas import tpu_sc as plsc

N_CLASSES = 256
TOTAL_BINS = 8
LANES = 16

NUM_CORES = 2
NUM_SUBCORES = 16
NUM_WORKERS = NUM_CORES * NUM_SUBCORES

CHUNK = 6400
VECS = CHUNK // LANES


def _make_sc_lookup(n_idx: int):
    assert n_idx % (NUM_WORKERS * CHUNK) == 0
    per_w = n_idx // NUM_WORKERS
    n_chunks = per_w // CHUNK
    assert n_chunks % 2 == 0

    mesh = plsc.VectorSubcoreMesh(
        core_axis_name="c", subcore_axis_name="s",
        num_cores=NUM_CORES, num_subcores=NUM_SUBCORES)

    @functools.partial(
        pl.kernel,
        out_type=jax.ShapeDtypeStruct((n_idx * TOTAL_BINS,), jnp.float32),
        mesh=mesh,
        scratch_types=[
            pltpu.VMEM((N_CLASSES * TOTAL_BINS,), jnp.float32),
            pltpu.VMEM((CHUNK,), jnp.int32),
            pltpu.VMEM((CHUNK,), jnp.int32),
            pltpu.VMEM((CHUNK * TOTAL_BINS,), jnp.float32),
            pltpu.VMEM((CHUNK * TOTAL_BINS,), jnp.float32),
            pltpu.SemaphoreType.DMA,
            pltpu.SemaphoreType.DMA,
            pltpu.SemaphoreType.DMA,
            pltpu.SemaphoreType.DMA,
        ],
        compiler_params=pltpu.CompilerParams(needs_layout_passes=False),
    )
    def lookup(table_hbm, idx_hbm, out_hbm, table_v,
               idx_v0, idx_v1, out_v0, out_v1,
               sem_i0, sem_i1, sem_o0, sem_o1):
        wid = lax.axis_index("s") * NUM_CORES + lax.axis_index("c")
        base = wid * per_w
        pltpu.sync_copy(table_hbm, table_v)
        lane8 = lax.iota(jnp.int32, LANES) * TOTAL_BINS

        idx_bufs = (idx_v0, idx_v1)
        out_bufs = (out_v0, out_v1)
        isems = (sem_i0, sem_i1)
        osems = (sem_o0, sem_o1)

        def idx_start(c):
            b = c % 2
            return pltpu.async_copy(
                idx_hbm.at[pl.ds(base + c * CHUNK, CHUNK)],
                idx_bufs[b], isems[b])

        def out_start(c):
            b = c % 2
            return pltpu.async_copy(
                out_bufs[b],
                out_hbm.at[pl.ds((base + c * CHUNK) * TOTAL_BINS,
                                 CHUNK * TOTAL_BINS)],
                osems[b])

        def compute(idx_b, out_b):
            @plsc.parallel_loop(0, VECS, 1, unroll=4)
            def body(vi):
                idxv = idx_b[pl.ds(vi * LANES, LANES)]
                srcs = idxv * TOTAL_BINS
                dsts = vi * (LANES * TOTAL_BINS) + lane8
                for j in range(TOTAL_BINS):
                    col = plsc.load_gather(table_v, [srcs + j])
                    plsc.store_scatter(out_b, [dsts + j], col)

        handles = {}
        handles["i0"] = idx_start(0)
        handles["i1"] = idx_start(1)
        for c in range(n_chunks):
            b = c % 2
            handles[f"i{c}"].wait()
            if c >= 2:
                handles[f"o{c - 2}"].wait()
            compute(idx_bufs[b], out_bufs[b])
            handles[f"o{c}"] = out_start(c)
            if c + 2 < n_chunks:
                handles[f"i{c + 2}"] = idx_start(c + 2)
        handles[f"o{n_chunks - 2}"].wait()
        handles[f"o{n_chunks - 1}"].wait()

    return lookup


def kernel(codebook, idx):
    n_idx = idx.size
    flat = _make_sc_lookup(n_idx)(codebook.reshape(-1), idx.reshape(-1))
    return flat.reshape(idx.shape + (2, TOTAL_BINS // 2))


# speedup vs baseline: 2.1601x; 2.1601x over previous
import functools

import jax
import jax.numpy as jnp
from jax import lax
from jax.experimental import pallas as pl
from jax.experimental.pallas import tpu as pltpu
from jax.experimental.pallas import tpu_sc as plsc

N_CLASSES = 256
TOTAL_BINS = 8

NUM_CORES = 2
NUM_SUBCORES = 16
NUM_WORKERS = NUM_CORES * NUM_SUBCORES

IDX_MINOR = 128
ROWS = 16
CHUNK = ROWS * IDX_MINOR


def _make_sc_lookup(n_idx: int):
    assert n_idx % (NUM_WORKERS * CHUNK) == 0
    per_w = n_idx // NUM_WORKERS
    n_chunks = per_w // CHUNK

    mesh = plsc.VectorSubcoreMesh(
        core_axis_name="c", subcore_axis_name="s",
        num_cores=NUM_CORES, num_subcores=NUM_SUBCORES)

    @functools.partial(
        pl.kernel,
        out_type=jax.ShapeDtypeStruct((n_idx, TOTAL_BINS), jnp.float32),
        mesh=mesh,
        scratch_types=[
            pltpu.VMEM((CHUNK,), jnp.int32),
            pltpu.VMEM((CHUNK, TOTAL_BINS), jnp.float32),
            pltpu.SemaphoreType.DMA,
        ],
        compiler_params=pltpu.CompilerParams(use_tc_tiling_on_sc=False),
    )
    def lookup(table_hbm, idx_hbm, out_hbm, idx_v, rows_v, sem):
        wid = lax.axis_index("s") * NUM_CORES + lax.axis_index("c")
        base = wid * per_w

        def chunk_body(ci, carry):
            off = base + ci * CHUNK
            pltpu.sync_copy(idx_hbm.at[pl.ds(off, CHUNK)], idx_v)
            pltpu.async_copy(table_hbm.at[idx_v], rows_v, sem).wait()
            pltpu.sync_copy(rows_v, out_hbm.at[pl.ds(off, CHUNK)])
            return carry

        lax.fori_loop(0, n_chunks, chunk_body, 0)

    return lookup


def kernel(codebook, idx):
    n_idx = idx.size
    rows = _make_sc_lookup(n_idx)(codebook, idx.reshape(n_idx))
    return rows.reshape(idx.shape + (2, TOTAL_BINS // 2))
